# Initial kernel scaffold; baseline (speedup 1.0000x reference)
import jax
import jax.numpy as jnp
from jax.experimental import pallas as pl


def kernel(x, Wg, We, be):
    raise NotImplementedError("write your pallas kernel here")


# speedup vs baseline: 1.8400x; 1.8400x over previous
import functools

import jax
import jax.numpy as jnp
from jax.experimental import pallas as pl

B, T, D, E, TOP_K = 2, 2048, 1024, 8, 2
ROWS = B * T
TILE = 256


def _moe_kernel(x_ref, wg_ref, we_ref, be_ref, out_ref, gate_ref):
    x = x_ref[...]
    xb = x.astype(jnp.bfloat16)

    wg = wg_ref[...].astype(jnp.bfloat16)
    logits = jax.lax.dot_general(
        xb, wg, (((1,), (1,)), ((), ())),
        preferred_element_type=jnp.float32)

    m = jnp.max(logits, axis=1, keepdims=True)
    eg = jnp.exp(logits - m)
    gate = eg / jnp.sum(eg, axis=1, keepdims=True)

    col = jax.lax.broadcasted_iota(jnp.int32, (TILE, E), 1)
    a1 = jnp.argmax(gate, axis=1)[:, None]
    w1 = jnp.max(gate, axis=1)[:, None]
    masked = jnp.where(col == a1, -jnp.inf, gate)
    a2 = jnp.argmax(masked, axis=1)[:, None]
    w2 = jnp.max(masked, axis=1)[:, None]

    acc = jnp.zeros((TILE, D), jnp.float32)
    for i in range(E):
        wi = w1 * (a1 == i) + w2 * (a2 == i)
        y = jax.lax.dot_general(
            xb, we_ref[i], (((1,), (1,)), ((), ())),
            preferred_element_type=jnp.float32)
        acc = acc + wi * (y + be_ref[i][None, :])

    out_ref[...] = jnp.maximum(acc, 0.0)
    gate_ref[...] = gate


@jax.jit
def kernel(x, Wg, We, be):
    x2 = x.reshape(ROWS, D)
    we_bf16 = We.astype(jnp.bfloat16)
    grid = (ROWS // TILE,)
    out, gate = pl.pallas_call(
        _moe_kernel,
        grid=grid,
        in_specs=[
            pl.BlockSpec((TILE, D), lambda i: (i, 0)),
            pl.BlockSpec((E, D), lambda i: (0, 0)),
            pl.BlockSpec((E, D, D), lambda i: (0, 0, 0)),
            pl.BlockSpec((E, D), lambda i: (0, 0)),
        ],
        out_specs=[
            pl.BlockSpec((TILE, D), lambda i: (i, 0)),
            pl.BlockSpec((TILE, E), lambda i: (i, 0)),
        ],
        out_shape=[
            jax.ShapeDtypeStruct((ROWS, D), jnp.float32),
            jax.ShapeDtypeStruct((ROWS, E), jnp.float32),
        ],
    )(x2, Wg, we_bf16, be)
    return out.reshape(B, T, D), gate.reshape(B, T, E)
